# Initial kernel scaffold; baseline (speedup 1.0000x reference)
import jax
import jax.numpy as jnp
from jax.experimental import pallas as pl


def kernel(boxes_pred, conf_pred, kpts_pred, dpth_pred, label_t, boxes_t, kypts_t, dpths_t):
    raise NotImplementedError("write your pallas kernel here")


# speedup vs baseline: 10.9310x; 10.9310x over previous
import jax
import jax.numpy as jnp
from jax import lax
from jax.experimental import pallas as pl
from jax.experimental.pallas import tpu as pltpu

_B = 32
_P = 16800
_BBLK = 8
_GRID = _B // _BBLK


def _smooth_l1(d):
    ad = jnp.abs(d)
    return jnp.where(ad < 1.0, 0.5 * d * d, ad - 0.5)


def _body(lab_ref, c0_ref, c1_ref,
          bp0, bp1, bp2, bp3, bt0, bt1, bt2, bt3,
          kp0, kp1, kp2, kp3, kp4, kp5, kp6, kp7, kp8, kp9,
          kt0, kt1, kt2, kt3, kt4, kt5, kt6, kt7, kt8, kt9,
          dp0, dp1, dt0, dt1,
          o_l, o_c, o_m, o_d,
          lm_scr, np_scr, acc):
    i = pl.program_id(0)

    @pl.when(i == 0)
    def _init():
        acc[0] = 0.0
        acc[1] = 0.0
        acc[2] = 0.0
        acc[3] = 0.0

    lab = lab_ref[...]
    pos = lab != 0
    posf = pos.astype(jnp.float32)

    c0 = c0_ref[...]
    c1 = c1_ref[...]
    m = jnp.maximum(c0, c1)
    lse = jnp.log(jnp.exp(c0 - m) + jnp.exp(c1 - m)) + m
    g = jnp.where(pos, c1, c0)
    ce = lse - g
    lm = jnp.where(pos, 0.0, ce)
    lm_scr[pl.ds(i * _BBLK, _BBLK), :] = lm
    np_scr[pl.ds(i * _BBLK, _BBLK), :] = jnp.sum(posf, axis=1, keepdims=True)
    acc[3] += jnp.sum(ce * posf)

    s_box = jnp.sum((_smooth_l1(bp0[...] - bt0[...]) + _smooth_l1(bp1[...] - bt1[...])
                     + _smooth_l1(bp2[...] - bt2[...]) + _smooth_l1(bp3[...] - bt3[...])) * posf)
    s_kpt = jnp.sum((_smooth_l1(kp0[...] - kt0[...]) + _smooth_l1(kp1[...] - kt1[...])
                     + _smooth_l1(kp2[...] - kt2[...]) + _smooth_l1(kp3[...] - kt3[...])
                     + _smooth_l1(kp4[...] - kt4[...]) + _smooth_l1(kp5[...] - kt5[...])
                     + _smooth_l1(kp6[...] - kt6[...]) + _smooth_l1(kp7[...] - kt7[...])
                     + _smooth_l1(kp8[...] - kt8[...]) + _smooth_l1(kp9[...] - kt9[...])) * posf)
    s_dpt = jnp.sum((_smooth_l1(dp0[...] - dt0[...]) + _smooth_l1(dp1[...] - dt1[...])) * posf)
    acc[0] += s_box
    acc[1] += s_kpt
    acc[2] += s_dpt

    @pl.when(i == _GRID - 1)
    def _finalize():
        lm_all = lm_scr[...]
        npv = np_scr[...]
        kf = jnp.minimum(npv * 7.0, float(_P - 1))
        ki = kf.astype(jnp.int32)
        keys = lax.bitcast_convert_type(lm_all, jnp.int32)

        def bis(_, lr):
            lo, hi = lr
            mid = lo + lax.shift_right_logical(hi - lo + 1, 1)
            cnt = jnp.sum((keys >= mid).astype(jnp.int32), axis=1, keepdims=True)
            ok = cnt >= ki
            return jnp.where(ok, mid, lo), jnp.where(ok, hi, mid - 1)

        lo0 = jnp.zeros_like(ki)
        hi0 = jnp.full_like(ki, 0x7F800000)
        lo, _ = lax.fori_loop(0, 31, bis, (lo0, hi0))
        t = lax.bitcast_convert_type(lo, jnp.float32)
        gt = (lm_all > t).astype(jnp.float32)
        cnt_gt = jnp.sum(gt, axis=1, keepdims=True)
        sum_gt = jnp.sum(lm_all * gt, axis=1, keepdims=True)
        t_safe = jnp.where(ki > 0, t, 0.0)
        t_row = jnp.where(ki > 0, sum_gt + (kf - cnt_gt) * t_safe, 0.0)

        npt = jnp.sum(npv)
        loss_c_sum = acc[3] + jnp.sum(t_row)
        o_l[...] = (acc[0] / jnp.maximum(npt * 4.0, 1.0)).reshape(1, 1)
        o_c[...] = (loss_c_sum / jnp.maximum(npt, 1.0)).reshape(1, 1)
        o_m[...] = (acc[1] / jnp.maximum(npt * 10.0, 1.0)).reshape(1, 1)
        o_d[...] = (acc[2] / jnp.maximum(npt * 2.0, 1.0)).reshape(1, 1)


def _impl(lab, c0, c1, box_ch, kpt_ch, dpt_ch, interpret=False):
    row_spec = pl.BlockSpec((_BBLK, _P), lambda i: (i, 0))
    out_spec = pl.BlockSpec((1, 1), lambda i: (0, 0))
    outs = pl.pallas_call(
        _body,
        grid=(_GRID,),
        in_specs=[row_spec] * 35,
        out_specs=[out_spec] * 4,
        out_shape=[jax.ShapeDtypeStruct((1, 1), jnp.float32)] * 4,
        scratch_shapes=[
            pltpu.VMEM((_B, _P), jnp.float32),
            pltpu.VMEM((_B, 1), jnp.float32),
            pltpu.SMEM((4,), jnp.float32),
        ],
        interpret=interpret,
    )(lab, c0, c1, *box_ch, *kpt_ch, *dpt_ch)
    return tuple(o.reshape(()) for o in outs)


def kernel(boxes_pred, conf_pred, kpts_pred, dpth_pred, label_t, boxes_t, kypts_t, dpths_t):
    lab = label_t.astype(jnp.int32)
    c0 = conf_pred[:, :, 0]
    c1 = conf_pred[:, :, 1]
    box_ch = [boxes_pred[:, :, k] for k in range(4)] + [boxes_t[:, :, k] for k in range(4)]
    kpt_ch = [kpts_pred[:, :, k] for k in range(10)] + [kypts_t[:, :, k] for k in range(10)]
    dpt_ch = [dpth_pred[:, :, k] for k in range(2)] + [dpths_t[:, :, k] for k in range(2)]
    return _impl(lab, c0, c1, box_ch, kpt_ch, dpt_ch)


# speedup vs baseline: 18.2329x; 1.6680x over previous
import jax
import jax.numpy as jnp
from jax import lax
from jax.experimental import pallas as pl
from jax.experimental.pallas import tpu as pltpu

_B = 32
_P = 16800
_BBLK = 8
_GRID = _B // _BBLK


def _smooth_l1(d):
    ad = jnp.abs(d)
    return jnp.where(ad < 1.0, 0.5 * d * d, ad - 0.5)


def _sl1_sum(pred, tgt):
    return jnp.sum(_smooth_l1(pred - tgt), axis=0)


def _body(lab_ref, conf_ref, bp_ref, bt_ref, kp_ref, kt_ref, dp_ref, dt_ref,
          o_l, o_c, o_m, o_d,
          lm_scr, np_scr, acc):
    i = pl.program_id(0)

    @pl.when(i == 0)
    def _init():
        acc[0] = 0.0
        acc[1] = 0.0
        acc[2] = 0.0
        acc[3] = 0.0

    lab = lab_ref[...]
    pos = lab != 0
    posf = pos.astype(jnp.float32)

    c0 = conf_ref[0]
    c1 = conf_ref[1]
    m = jnp.maximum(c0, c1)
    lse = jnp.log(jnp.exp(c0 - m) + jnp.exp(c1 - m)) + m
    g = jnp.where(pos, c1, c0)
    ce = lse - g
    lm = jnp.where(pos, 0.0, ce)
    lm_scr[pl.ds(i * _BBLK, _BBLK), :] = lm
    np_scr[pl.ds(i * _BBLK, _BBLK), :] = jnp.sum(posf, axis=1, keepdims=True)
    acc[3] += jnp.sum(ce * posf)

    acc[0] += jnp.sum(_sl1_sum(bp_ref[...], bt_ref[...]) * posf)
    acc[1] += jnp.sum(_sl1_sum(kp_ref[...], kt_ref[...]) * posf)
    acc[2] += jnp.sum(_sl1_sum(dp_ref[...], dt_ref[...]) * posf)

    @pl.when(i == _GRID - 1)
    def _finalize():
        lm_all = lm_scr[...]
        npv = np_scr[...]
        kf = jnp.minimum(npv * 7.0, float(_P - 1))
        ki = kf.astype(jnp.int32)
        keys = lax.bitcast_convert_type(lm_all, jnp.int32)

        def bis(_, lr):
            lo, hi = lr
            mid = lo + lax.shift_right_logical(hi - lo + 1, 1)
            cnt = jnp.sum((keys >= mid).astype(jnp.int32), axis=1, keepdims=True)
            ok = cnt >= ki
            return jnp.where(ok, mid, lo), jnp.where(ok, hi, mid - 1)

        lo0 = jnp.zeros_like(ki)
        hi0 = jnp.full_like(ki, 0x7F800000)
        lo, _ = lax.fori_loop(0, 31, bis, (lo0, hi0))
        t = lax.bitcast_convert_type(lo, jnp.float32)
        gt = (lm_all > t).astype(jnp.float32)
        cnt_gt = jnp.sum(gt, axis=1, keepdims=True)
        sum_gt = jnp.sum(lm_all * gt, axis=1, keepdims=True)
        t_safe = jnp.where(ki > 0, t, 0.0)
        t_row = jnp.where(ki > 0, sum_gt + (kf - cnt_gt) * t_safe, 0.0)

        npt = jnp.sum(npv)
        loss_c_sum = acc[3] + jnp.sum(t_row)
        o_l[...] = (acc[0] / jnp.maximum(npt * 4.0, 1.0)).reshape(1, 1)
        o_c[...] = (loss_c_sum / jnp.maximum(npt, 1.0)).reshape(1, 1)
        o_m[...] = (acc[1] / jnp.maximum(npt * 10.0, 1.0)).reshape(1, 1)
        o_d[...] = (acc[2] / jnp.maximum(npt * 2.0, 1.0)).reshape(1, 1)


def _impl(lab, conf_t, bp_t, bt_t, kp_t, kt_t, dp_t, dt_t, interpret=False):
    def kspec(k):
        return pl.BlockSpec((k, _BBLK, _P), lambda i: (0, i, 0))

    row_spec = pl.BlockSpec((_BBLK, _P), lambda i: (i, 0))
    out_spec = pl.BlockSpec((1, 1), lambda i: (0, 0))
    outs = pl.pallas_call(
        _body,
        grid=(_GRID,),
        in_specs=[row_spec, kspec(2), kspec(4), kspec(4), kspec(10), kspec(10),
                  kspec(2), kspec(2)],
        out_specs=[out_spec] * 4,
        out_shape=[jax.ShapeDtypeStruct((1, 1), jnp.float32)] * 4,
        scratch_shapes=[
            pltpu.VMEM((_B, _P), jnp.float32),
            pltpu.VMEM((_B, 1), jnp.float32),
            pltpu.SMEM((4,), jnp.float32),
        ],
        interpret=interpret,
    )(lab, conf_t, bp_t, bt_t, kp_t, kt_t, dp_t, dt_t)
    return tuple(o.reshape(()) for o in outs)


def kernel(boxes_pred, conf_pred, kpts_pred, dpth_pred, label_t, boxes_t, kypts_t, dpths_t):
    lab = label_t.astype(jnp.int32)
    tr = lambda x: jnp.transpose(x, (2, 0, 1))
    return _impl(lab, tr(conf_pred), tr(boxes_pred), tr(boxes_t),
                 tr(kpts_pred), tr(kypts_t), tr(dpth_pred), tr(dpths_t))
